# baseline (speedup 1.0000x reference)
import jax
import jax.numpy as jnp
from jax.experimental import pallas as pl

_TEMPERATURE = 100.0
_THRESHOLD = 0.3
_TOPK = 30


def _corr_kernel(fr_ref, fc_ref, fm_ref, out_ref):
    fr = fr_ref[0]
    fc = fc_ref[0]
    fm = fm_ref[0]

    frn = fr / (jnp.sqrt(jnp.sum(fr * fr, axis=-1, keepdims=True)) + 1e-6)
    fcn = fc / (jnp.sqrt(jnp.sum(fc * fc, axis=-1, keepdims=True)) + 1e-6)

    g = jax.lax.dot_general(
        frn, fcn, (((1,), (1,)), ((), ())),
        preferred_element_type=jnp.float32)

    fmaskf = (fm > _THRESHOLD).astype(jnp.float32)
    logits = (g / _TEMPERATURE) * fmaskf
    e = jnp.exp(logits)
    s = jnp.sum(e, axis=-1, keepdims=True)
    x = e / s

    m = jnp.maximum(x[:, :512], x[:, 512:])
    m = jnp.maximum(m[:, :256], m[:, 256:])
    m = jnp.maximum(m[:, :128], m[:, 128:])
    m64 = jnp.maximum(m[:, :64], m[:, 64:])
    m = jnp.maximum(m64[:, :32], m64[:, 32:])

    mn = jnp.min(m, axis=-1, keepdims=True)
    eqmn = m == mn
    cmn = jnp.sum(eqmn.astype(jnp.float32), axis=-1, keepdims=True)
    mn2 = jnp.min(jnp.where(eqmn, 2.0, m), axis=-1, keepdims=True)
    lo0 = jnp.where(cmn >= 2.0, mn, mn2)

    mx = jnp.max(m64, axis=-1, keepdims=True)
    eqmx = m64 == mx
    cmx = jnp.sum(eqmx.astype(jnp.float32), axis=-1, keepdims=True)
    mx2 = jnp.max(jnp.where(eqmx, -1.0, m64), axis=-1, keepdims=True)
    u = jnp.where(cmx >= 2.0, mx, mx2)
    hi0 = u * (1.0 + 3e-7)

    ones_k = jnp.ones((1, x.shape[1]), jnp.float32)
    target = float(_TOPK + 1)

    def count_at(mid):
        mask = (x >= mid).astype(jnp.float32)
        return jax.lax.dot_general(
            mask, ones_k, (((1,), (1,)), ((), ())),
            preferred_element_type=jnp.float32)

    def probe(carry, mid):
        lo, hi = carry
        cnt = count_at(mid)
        ge = cnt >= target
        return (jnp.where(ge, mid, lo), jnp.where(ge, hi, mid)), cnt

    def body(carry):
        lo, hi = carry
        carry, _ = probe(carry, 0.5 * (lo + hi))
        return carry

    def cond(carry):
        lo, hi = carry
        mid = 0.5 * (lo + hi)
        return jnp.any((mid > lo) & (mid < hi))

    carry = (lo0, hi0)
    mid1 = 0.5 * (carry[0] + carry[1])
    carry, c1 = probe(carry, mid1)
    mid2 = 0.5 * (carry[0] + carry[1])
    carry, c2 = probe(carry, mid2)
    lt = jnp.log(target)
    vprev, lprev = mid1, jnp.log(jnp.maximum(c1, 0.5))
    vcur, lcur = mid2, jnp.log(jnp.maximum(c2, 0.5))
    for _ in range(2):
        lo, hi = carry
        denom = lcur - lprev
        safe = jnp.abs(denom) > 1e-6
        v = vcur + (lt - lcur) * (vcur - vprev) / jnp.where(safe, denom, 1.0)
        w = hi - lo
        v = jnp.clip(v, lo + 0.0625 * w, hi - 0.0625 * w)
        v = jnp.where(safe & (v > lo) & (v < hi), v, 0.5 * (lo + hi))
        carry, c = probe(carry, v)
        vprev, lprev = vcur, lcur
        vcur, lcur = v, jnp.log(jnp.maximum(c, 0.5))
    for _ in range(7):
        carry = body(carry)
    thresh, _ = jax.lax.while_loop(cond, body, carry)

    out_ref[0] = jnp.where(x > thresh, x, 0.0)


def kernel(feat_ref, feat_cur, filter_mask, topk):
    del topk
    b, q, d = feat_ref.shape
    k = feat_cur.shape[1]
    return pl.pallas_call(
        _corr_kernel,
        grid=(b,),
        in_specs=[
            pl.BlockSpec((1, q, d), lambda i: (i, 0, 0)),
            pl.BlockSpec((1, k, d), lambda i: (i, 0, 0)),
            pl.BlockSpec((1, 1, k), lambda i: (i, 0, 0)),
        ],
        out_specs=pl.BlockSpec((1, q, k), lambda i: (i, 0, 0)),
        out_shape=jax.ShapeDtypeStruct((b, q, k), jnp.float32),
    )(feat_ref, feat_cur, filter_mask.reshape(b, 1, k))


# speedup vs baseline: 1.2055x; 1.2055x over previous
import jax
import jax.numpy as jnp
from jax.experimental import pallas as pl

_TEMPERATURE = 100.0
_THRESHOLD = 0.3
_TOPK = 30


def _corr_kernel(fr_ref, fc_ref, fm_ref, out_ref):
    fr = fr_ref[0]
    fc = fc_ref[0]
    fm = fm_ref[0]

    frn = fr / (jnp.sqrt(jnp.sum(fr * fr, axis=-1, keepdims=True)) + 1e-6)
    fcn = fc / (jnp.sqrt(jnp.sum(fc * fc, axis=-1, keepdims=True)) + 1e-6)

    g = jax.lax.dot_general(
        frn, fcn, (((1,), (1,)), ((), ())),
        preferred_element_type=jnp.float32)

    fmaskf = (fm > _THRESHOLD).astype(jnp.float32)
    logits = (g / _TEMPERATURE) * fmaskf
    e = jnp.exp(logits)
    s = jnp.sum(e, axis=-1, keepdims=True)
    x = e / s

    m = jnp.maximum(x[:, :512], x[:, 512:])
    m = jnp.maximum(m[:, :256], m[:, 256:])
    m = jnp.maximum(m[:, :128], m[:, 128:])
    m64 = jnp.maximum(m[:, :64], m[:, 64:])
    m = jnp.maximum(m64[:, :32], m64[:, 32:])

    mn = jnp.min(m, axis=-1, keepdims=True)
    eqmn = m == mn
    cmn = jnp.sum(eqmn.astype(jnp.float32), axis=-1, keepdims=True)
    mn2 = jnp.min(jnp.where(eqmn, 2.0, m), axis=-1, keepdims=True)
    lo0 = jnp.where(cmn >= 2.0, mn, mn2)

    mx = jnp.max(m64, axis=-1, keepdims=True)
    eqmx = m64 == mx
    cmx = jnp.sum(eqmx.astype(jnp.float32), axis=-1, keepdims=True)
    mx2 = jnp.max(jnp.where(eqmx, -1.0, m64), axis=-1, keepdims=True)
    u = jnp.where(cmx >= 2.0, mx, mx2)
    hi0 = u * (1.0 + 3e-7)

    ones_k = jnp.ones((1, x.shape[1]), jnp.float32)

    def body(carry):
        lo, hi = carry
        mid = 0.5 * (lo + hi)
        mask = (x >= mid).astype(jnp.float32)
        cnt = jax.lax.dot_general(
            mask, ones_k, (((1,), (1,)), ((), ())),
            preferred_element_type=jnp.float32)
        ge = cnt >= float(_TOPK + 1)
        return (jnp.where(ge, mid, lo), jnp.where(ge, hi, mid))

    def cond(carry):
        lo, hi = carry
        mid = 0.5 * (lo + hi)
        return jnp.any((mid > lo) & (mid < hi))

    carry = (lo0, hi0)
    for _ in range(14):
        carry = body(carry)
    thresh, _ = jax.lax.while_loop(cond, body, carry)

    out_ref[0] = jnp.where(x > thresh, x, 0.0)


def kernel(feat_ref, feat_cur, filter_mask, topk):
    del topk
    b, q, d = feat_ref.shape
    k = feat_cur.shape[1]
    return pl.pallas_call(
        _corr_kernel,
        grid=(b,),
        in_specs=[
            pl.BlockSpec((1, q, d), lambda i: (i, 0, 0)),
            pl.BlockSpec((1, k, d), lambda i: (i, 0, 0)),
            pl.BlockSpec((1, 1, k), lambda i: (i, 0, 0)),
        ],
        out_specs=pl.BlockSpec((1, q, k), lambda i: (i, 0, 0)),
        out_shape=jax.ShapeDtypeStruct((b, q, k), jnp.float32),
    )(feat_ref, feat_cur, filter_mask.reshape(b, 1, k))
